# baseline (speedup 1.0000x reference)
import functools

import jax
import jax.numpy as jnp
from jax import lax
from jax.experimental import pallas as pl
from jax.experimental.pallas import tpu as pltpu
from jax.experimental.pallas import tpu_sc as plsc

M = 65536
D = 128
B = 16384
NC = 2
NS = 16
NW = NC * NS
R = M // NW
L = 16
CHUNK = 128
NCH = R // CHUNK
COPY_ROWS = 512
POS_SHIFT = 14


def _body(bank_hbm, idx_hbm, val_hbm, out_hbm,
          idx_v, t_v, loc2, pos2, rows_v, copy_buf):
    wid = lax.axis_index("s") * NC + lax.axis_index("c")
    base = wid * R
    iota = lax.iota(jnp.int32, L)

    def copy_body(c, _):
        r0 = base + c * COPY_ROWS
        pltpu.sync_copy(bank_hbm.at[pl.ds(r0, COPY_ROWS)], copy_buf)
        pltpu.sync_copy(copy_buf, out_hbm.at[pl.ds(r0, COPY_ROWS)])
        return _
    lax.fori_loop(0, R // COPY_ROWS, copy_body, 0)

    pltpu.sync_copy(idx_hbm, idx_v)

    minus1 = jnp.full((L,), -1, jnp.int32)

    def init_body(i, _):
        t_v[pl.ds(i * L, L)] = minus1
        return _
    lax.fori_loop(0, R // L, init_body, 0)

    def scan_body(v, _):
        g = idx_v[pl.ds(v * L, L)]
        pos = v * L + iota
        key = (g << POS_SHIFT) | pos
        ks, ps = plsc.sort_key_val(key, pos)
        slot = lax.shift_right_logical(ks, POS_SHIFT)
        m_in = (slot >= base) & (slot < base + R)
        nxt = slot.at[jnp.minimum(iota + 1, L - 1)].get(
            mode="promise_in_bounds")
        keep = (slot != nxt) | (iota == L - 1)
        plsc.store_scatter(t_v, [slot - base], ps, mask=m_in & keep)
        return _
    lax.fori_loop(0, B // L, scan_body, 0)

    def compact_body(i, cnt):
        t = t_v[pl.ds(i * L, L)]
        m = t >= 0
        cs = plsc.cumsum(m.astype(jnp.int32))
        dest = cnt + cs - 1
        drow = lax.shift_right_logical(dest, 7)
        dcol = dest & (CHUNK - 1)
        gslot = base + i * L + iota
        plsc.store_scatter(loc2, [drow, dcol], gslot, mask=m)
        plsc.store_scatter(pos2, [drow, dcol], t, mask=m)
        return cnt + plsc.all_reduce_population_count(m)
    cnt = lax.fori_loop(0, R // L, compact_body, jnp.zeros((L,), jnp.int32))
    n = jnp.max(cnt)

    @pl.when(n > 0)
    def _():
        nch = (n + CHUNK - 1) // CHUNK
        zeros = jnp.zeros((L,), jnp.int32)
        e_loc = loc2[0, pl.ds(0, L)].at[zeros].get(mode="promise_in_bounds")
        e_pos = pos2[0, pl.ds(0, L)].at[zeros].get(mode="promise_in_bounds")
        for k in range(CHUNK // L):
            gidx = (nch - 1) * CHUNK + k * L + iota
            mpad = gidx >= n
            grow = lax.shift_right_logical(gidx, 7)
            gcol = gidx & (CHUNK - 1)
            plsc.store_scatter(loc2, [grow, gcol], e_loc, mask=mpad)
            plsc.store_scatter(pos2, [grow, gcol], e_pos, mask=mpad)

        def chunk_body(j, _):
            pltpu.sync_copy(val_hbm.at[pos2.at[j]], rows_v)
            pltpu.sync_copy(rows_v, out_hbm.at[loc2.at[j]])
            return _
        lax.fori_loop(0, nch, chunk_body, 0)


@jax.jit
def _scatter_set(bank, idx, val):
    mesh = plsc.VectorSubcoreMesh(core_axis_name="c", subcore_axis_name="s")
    f = functools.partial(
        pl.kernel,
        mesh=mesh,
        compiler_params=pltpu.CompilerParams(needs_layout_passes=False),
        out_type=jax.ShapeDtypeStruct((M, D), jnp.float32),
        scratch_types=[
            pltpu.VMEM((B,), jnp.int32),
            pltpu.VMEM((R,), jnp.int32),
            pltpu.VMEM((NCH, CHUNK), jnp.int32),
            pltpu.VMEM((NCH, CHUNK), jnp.int32),
            pltpu.VMEM((CHUNK, D), jnp.float32),
            pltpu.VMEM((COPY_ROWS, D), jnp.float32),
        ],
    )(_body)
    return f(bank, idx, val)


def kernel(skill_embeddings, idx, val):
    return _scatter_set(skill_embeddings, idx, val)


# speedup vs baseline: 1.0363x; 1.0363x over previous
import functools

import jax
import jax.numpy as jnp
from jax import lax
from jax.experimental import pallas as pl
from jax.experimental.pallas import tpu as pltpu
from jax.experimental.pallas import tpu_sc as plsc

M = 65536
D = 128
B = 16384
NC = 2
NS = 16
NW = NC * NS
R = M // NW
RSHIFT = 11
L = 16
CHUNK = 128
NCH = R // CHUNK
CB = 128
NCOPY = R // CB
NBUF = 3
POS_SHIFT = 14
SCAN_PER = (B // L) // NCOPY


def _body(bank_hbm, idx_hbm, val_hbm, out_hbm,
          idx_v, t_v, loc2, pos2, rows_v, cb0, cb1, cb2,
          sem_i, sem_r0, sem_r1, sem_r2, sem_w0, sem_w1, sem_w2):
    wid = lax.axis_index("s") * NC + lax.axis_index("c")
    base = wid * R
    iota = lax.iota(jnp.int32, L)
    cbufs = [cb0, cb1, cb2]
    sem_r = [sem_r0, sem_r1, sem_r2]
    sem_w = [sem_w0, sem_w1, sem_w2]

    icp = pltpu.async_copy(idx_hbm, idx_v, sem_i)

    def rd(c):
        b = c % NBUF
        return pltpu.async_copy(
            bank_hbm.at[pl.ds(base + c * CB, CB)], cbufs[b], sem_r[b])

    def wr(c):
        b = c % NBUF
        return pltpu.async_copy(
            cbufs[b], out_hbm.at[pl.ds(base + c * CB, CB)], sem_w[b])

    rds = {c: rd(c) for c in range(NBUF)}
    icp.wait()

    minus1 = jnp.full((L,), -1, jnp.int32)

    def init_body(i, _):
        t_v[pl.ds(i * L, L)] = minus1
        return _
    lax.fori_loop(0, R // L, init_body, 0)

    def scan_body(v, carry):
        g = idx_v[pl.ds(v * L, L)]
        hit = lax.shift_right_logical(g, RSHIFT) == wid

        @pl.when(jnp.any(hit))
        def _do_scan():
            pos = v * L + iota
            key = (g << POS_SHIFT) | pos
            ks, ps = plsc.sort_key_val(key, pos)
            m_in = lax.shift_right_logical(ks, POS_SHIFT + RSHIFT) == wid
            slot = lax.shift_right_logical(ks, POS_SHIFT)
            nxt = slot.at[jnp.minimum(iota + 1, L - 1)].get(
                mode="promise_in_bounds")
            keep = (slot != nxt) | (iota == L - 1)
            plsc.store_scatter(t_v, [slot - base], ps, mask=m_in & keep)
        return carry

    wrs = {}
    for c in range(NCOPY):
        rds[c].wait()
        wrs[c] = wr(c)
        lax.fori_loop(c * SCAN_PER, (c + 1) * SCAN_PER, scan_body, 0)
        if c + NBUF < NCOPY:
            wrs[c].wait()
            rds[c + NBUF] = rd(c + NBUF)
    for c in range(NCOPY - NBUF, NCOPY):
        wrs[c].wait()

    def compact_body(i, cnt):
        t = t_v[pl.ds(i * L, L)]
        m = t >= 0
        cs = plsc.cumsum(m.astype(jnp.int32))
        dest = cnt + cs - 1
        drow = lax.shift_right_logical(dest, 7)
        dcol = dest & (CHUNK - 1)
        gslot = base + i * L + iota
        plsc.store_scatter(loc2, [drow, dcol], gslot, mask=m)
        plsc.store_scatter(pos2, [drow, dcol], t, mask=m)
        return cnt + plsc.all_reduce_population_count(m)
    cnt = lax.fori_loop(0, R // L, compact_body, jnp.zeros((L,), jnp.int32))
    n = jnp.max(cnt)

    @pl.when(n > 0)
    def _():
        nch = (n + CHUNK - 1) // CHUNK
        zeros = jnp.zeros((L,), jnp.int32)
        e_loc = loc2[0, pl.ds(0, L)].at[zeros].get(mode="promise_in_bounds")
        e_pos = pos2[0, pl.ds(0, L)].at[zeros].get(mode="promise_in_bounds")
        for k in range(CHUNK // L):
            gidx = (nch - 1) * CHUNK + k * L + iota
            mpad = gidx >= n
            grow = lax.shift_right_logical(gidx, 7)
            gcol = gidx & (CHUNK - 1)
            plsc.store_scatter(loc2, [grow, gcol], e_loc, mask=mpad)
            plsc.store_scatter(pos2, [grow, gcol], e_pos, mask=mpad)

        def chunk_body(j, _):
            pltpu.sync_copy(val_hbm.at[pos2.at[j]], rows_v)
            pltpu.sync_copy(rows_v, out_hbm.at[loc2.at[j]])
            return _
        lax.fori_loop(0, nch, chunk_body, 0)


@jax.jit
def _scatter_set(bank, idx, val):
    mesh = plsc.VectorSubcoreMesh(core_axis_name="c", subcore_axis_name="s")
    f = functools.partial(
        pl.kernel,
        mesh=mesh,
        compiler_params=pltpu.CompilerParams(needs_layout_passes=False),
        out_type=jax.ShapeDtypeStruct((M, D), jnp.float32),
        scratch_types=[
            pltpu.VMEM((B,), jnp.int32),
            pltpu.VMEM((R,), jnp.int32),
            pltpu.VMEM((NCH, CHUNK), jnp.int32),
            pltpu.VMEM((NCH, CHUNK), jnp.int32),
            pltpu.VMEM((CHUNK, D), jnp.float32),
            pltpu.VMEM((CB, D), jnp.float32),
            pltpu.VMEM((CB, D), jnp.float32),
            pltpu.VMEM((CB, D), jnp.float32),
            pltpu.SemaphoreType.DMA,
            pltpu.SemaphoreType.DMA,
            pltpu.SemaphoreType.DMA,
            pltpu.SemaphoreType.DMA,
            pltpu.SemaphoreType.DMA,
            pltpu.SemaphoreType.DMA,
            pltpu.SemaphoreType.DMA,
        ],
    )(_body)
    return f(bank, idx, val)


def kernel(skill_embeddings, idx, val):
    return _scatter_set(skill_embeddings, idx, val)
